# baseline (speedup 1.0000x reference)
import jax
import jax.numpy as jnp
from jax import lax
from jax.experimental import pallas as pl
from jax.experimental.pallas import tpu as pltpu

N, C = 16384, 1000
BC = 2048
BR = 200
GC = N // BC
GR = C // BR


def _body(tgt_ref, rwd_ref, pt_ref, out_ref, acc_ref):
    c = pl.program_id(0)
    r = pl.program_id(1)
    tgt = tgt_ref[...]
    pb = pt_ref[...]
    rows = r * BR + lax.broadcasted_iota(jnp.int32, (BR, BC), 0)
    picked = jnp.sum(jnp.where(rows == tgt[None, :], pb, 0.0), axis=0)

    @pl.when(r == 0)
    def _():
        acc_ref[...] = jnp.zeros((BC,), jnp.float32)

    acc_ref[...] += picked

    @pl.when(jnp.logical_and(r == GR - 1, c == 0))
    def _():
        out_ref[0, 0] = 0.0

    @pl.when(r == GR - 1)
    def _():
        rwd = rwd_ref[...]
        out_ref[0, 0] += jnp.sum(acc_ref[...] * rwd) * (-1.0 / N)


@jax.jit
def _ganloss(pt, target, reward):
    out = pl.pallas_call(
        _body,
        grid=(GC, GR),
        in_specs=[
            pl.BlockSpec((BC,), lambda c, r: (c,)),
            pl.BlockSpec((BC,), lambda c, r: (c,)),
            pl.BlockSpec((BR, BC), lambda c, r: (r, c)),
        ],
        out_specs=pl.BlockSpec(
            (1, 1), lambda c, r: (0, 0), memory_space=pltpu.SMEM
        ),
        out_shape=jax.ShapeDtypeStruct((1, 1), jnp.float32),
        scratch_shapes=[pltpu.VMEM((BC,), jnp.float32)],
    )(target, reward, pt)
    return out[0, 0]


def kernel(prob, target, reward):
    return _ganloss(prob.T, target.astype(jnp.int32), reward)


# speedup vs baseline: 1.7112x; 1.7112x over previous
import jax
import jax.numpy as jnp
from jax import lax
from jax.experimental import pallas as pl
from jax.experimental.pallas import tpu as pltpu

N, C = 16384, 1000
BC = 2048
GRID = N // BC


def _body(tgt_ref, rwd_ref, pt_ref, out_ref):
    g = pl.program_id(0)
    tgt = tgt_ref[...]
    rwd = rwd_ref[...]
    pb = pt_ref[...]
    rows = lax.broadcasted_iota(jnp.int32, (C, BC), 0)
    picked = jnp.where(rows == tgt[None, :], pb, 0.0)
    partial = jnp.sum(jnp.sum(picked, axis=0) * rwd)

    @pl.when(g == 0)
    def _():
        out_ref[0, 0] = 0.0

    out_ref[0, 0] += partial * (-1.0 / N)


@jax.jit
def _ganloss(pt, target, reward):
    out = pl.pallas_call(
        _body,
        grid=(GRID,),
        in_specs=[
            pl.BlockSpec((BC,), lambda g: (g,)),
            pl.BlockSpec((BC,), lambda g: (g,)),
            pl.BlockSpec((C, BC), lambda g: (0, g)),
        ],
        out_specs=pl.BlockSpec(
            (1, 1), lambda g: (0, 0), memory_space=pltpu.SMEM
        ),
        out_shape=jax.ShapeDtypeStruct((1, 1), jnp.float32),
    )(target, reward, pt)
    return out[0, 0]


def kernel(prob, target, reward):
    return _ganloss(prob.T, target.astype(jnp.int32), reward)
